# Initial kernel scaffold; baseline (speedup 1.0000x reference)
import jax
import jax.numpy as jnp
from jax.experimental import pallas as pl


def kernel(sequence_data, masked_positions, candidate_sets, embedding_table, W, b, gamma, beta):
    raise NotImplementedError("write your pallas kernel here")


# speedup vs baseline: 2.0978x; 2.0978x over previous
import functools

import jax
import jax.numpy as jnp
from jax import lax
from jax.experimental import pallas as pl
from jax.experimental.pallas import tpu as pltpu
from jax.experimental.pallas import tpu_sc as plsc

B = 1024
S = 200
H = 128
P = 20
K = 50
E = 64

NPRED = B * P
K_PAD = 56
NC = 2
NS = 16
NW = NC * NS
PER_W = NPRED // NW
G = 2
NCHUNK = PER_W // G
IDX_N = G * K_PAD


BB = 8


def _tc_body(pos_ref, seq_ref, w_ref, b_ref, g_ref, be_ref, out_ref):
    pos = pos_ref[...]
    w = w_ref[...]
    rows = []
    for bb in range(BB):
        oh = (pos[bb][:, None] == lax.broadcasted_iota(jnp.int32, (P, S), 1))
        oh = oh.astype(jnp.float32)
        rows.append(jnp.dot(oh, seq_ref[bb], preferred_element_type=jnp.float32))
    x = jnp.concatenate(rows, axis=0)
    y = jnp.dot(x, w, preferred_element_type=jnp.float32) + b_ref[0]
    mean = jnp.mean(y, axis=1, keepdims=True)
    var = jnp.mean(jnp.square(y - mean), axis=1, keepdims=True)
    out = (y - mean) * lax.rsqrt(var + 1e-12) * g_ref[0] + be_ref[0]
    out_ref[...] = out.reshape(BB, P, E)


def _tc_lm(masked_positions, sequence_data, W, b, gamma, beta):
    grid = (B // BB,)
    return pl.pallas_call(
        _tc_body,
        grid=grid,
        in_specs=[
            pl.BlockSpec((BB, P), lambda i: (i, 0)),
            pl.BlockSpec((BB, S, H), lambda i: (i, 0, 0)),
            pl.BlockSpec((H, E), lambda i: (0, 0)),
            pl.BlockSpec((1, E), lambda i: (0, 0)),
            pl.BlockSpec((1, E), lambda i: (0, 0)),
            pl.BlockSpec((1, E), lambda i: (0, 0)),
        ],
        out_specs=pl.BlockSpec((BB, P, E), lambda i: (i, 0, 0)),
        out_shape=jax.ShapeDtypeStruct((B, P, E), jnp.float32),
    )(masked_positions, sequence_data, W,
      b.reshape(1, E), gamma.reshape(1, E), beta.reshape(1, E))


def _sc_body(emb_hbm, cand_hbm, lm_hbm, out_hbm, idx_v, rows_v, lm_v, out_v, sem):
    wid = lax.axis_index("s") * NC + lax.axis_index("c")
    w_base = wid * PER_W
    lanes = lax.iota(jnp.int32, 16)

    def chunk(i, carry):
        base = w_base + i * G
        pltpu.sync_copy(cand_hbm.at[pl.ds(base * K_PAD, IDX_N)], idx_v)
        gather = pltpu.async_copy(emb_hbm.at[idx_v], rows_v, sem)
        pltpu.sync_copy(lm_hbm.at[pl.ds(base, G)], lm_v)
        gather.wait()

        for g in range(G):
            l0 = lm_v[g, pl.ds(0, 16)]
            l1 = lm_v[g, pl.ds(16, 16)]
            l2 = lm_v[g, pl.ds(32, 16)]
            l3 = lm_v[g, pl.ds(48, 16)]
            for t in range(4):
                acc = jnp.zeros((16,), jnp.float32)
                for m in range(min(16, K - 16 * t)):
                    r = g * K_PAD + 16 * t + m
                    prod = (rows_v[r, pl.ds(0, 16)] * l0
                            + rows_v[r, pl.ds(16, 16)] * l1
                            + rows_v[r, pl.ds(32, 16)] * l2
                            + rows_v[r, pl.ds(48, 16)] * l3)
                    acc = jnp.where(lanes == m, jnp.sum(prod), acc)
                out_v[g, pl.ds(16 * t, 16)] = acc

        pltpu.sync_copy(out_v, out_hbm.at[pl.ds(base, G)])
        return carry

    lax.fori_loop(0, NCHUNK, chunk, 0)


def _sc_score(embedding_table, cand_flat, lm_flat):
    mesh = plsc.VectorSubcoreMesh(core_axis_name="c", subcore_axis_name="s")
    kern = functools.partial(
        pl.kernel,
        out_type=jax.ShapeDtypeStruct((NPRED, 64), jnp.float32),
        mesh=mesh,
        scratch_types=[
            pltpu.VMEM((IDX_N,), jnp.int32),
            pltpu.VMEM((IDX_N, E), jnp.float32),
            pltpu.VMEM((G, E), jnp.float32),
            pltpu.VMEM((G, 64), jnp.float32),
            pltpu.SemaphoreType.DMA,
        ],
        compiler_params=pltpu.CompilerParams(
            needs_layout_passes=False, use_tc_tiling_on_sc=False),
    )(_sc_body)
    return kern(embedding_table, cand_flat, lm_flat)


def kernel(sequence_data, masked_positions, candidate_sets, embedding_table, W, b, gamma, beta):
    lm = _tc_lm(masked_positions, sequence_data, W, b, gamma, beta)
    lm_flat = lm.reshape(NPRED, E)
    cand = candidate_sets.reshape(NPRED, K).astype(jnp.int32)
    cand_pad = jnp.concatenate([cand, cand[:, : K_PAD - K]], axis=1)
    out = _sc_score(embedding_table, cand_pad.reshape(-1), lm_flat)
    return out[:, :K].reshape(B, P, K)


# speedup vs baseline: 2.6772x; 1.2762x over previous
import functools

import jax
import jax.numpy as jnp
from jax import lax
from jax.experimental import pallas as pl
from jax.experimental.pallas import tpu as pltpu
from jax.experimental.pallas import tpu_sc as plsc

B = 1024
S = 200
H = 128
P = 20
K = 50
E = 64

NPRED = B * P
K_PAD = 56
NC = 2
NS = 16
NW = NC * NS
PER_W = NPRED // NW
G = 2
NCHUNK = PER_W // G
IDX_N = G * K_PAD


BB = 8


def _tc_body(pos_ref, seq_ref, w_ref, b_ref, g_ref, be_ref, out_ref):
    pos = pos_ref[...]
    w = w_ref[...]
    rows = []
    for bb in range(BB):
        oh = (pos[bb][:, None] == lax.broadcasted_iota(jnp.int32, (P, S), 1))
        oh = oh.astype(jnp.float32)
        rows.append(jnp.dot(oh, seq_ref[bb], preferred_element_type=jnp.float32))
    x = jnp.concatenate(rows, axis=0)
    y = jnp.dot(x, w, preferred_element_type=jnp.float32) + b_ref[0]
    mean = jnp.mean(y, axis=1, keepdims=True)
    var = jnp.mean(jnp.square(y - mean), axis=1, keepdims=True)
    out = (y - mean) * lax.rsqrt(var + 1e-12) * g_ref[0] + be_ref[0]
    out_ref[...] = out.reshape(BB, P, E)


def _tc_lm(masked_positions, sequence_data, W, b, gamma, beta):
    grid = (B // BB,)
    return pl.pallas_call(
        _tc_body,
        grid=grid,
        in_specs=[
            pl.BlockSpec((BB, P), lambda i: (i, 0)),
            pl.BlockSpec((BB, S, H), lambda i: (i, 0, 0)),
            pl.BlockSpec((H, E), lambda i: (0, 0)),
            pl.BlockSpec((1, E), lambda i: (0, 0)),
            pl.BlockSpec((1, E), lambda i: (0, 0)),
            pl.BlockSpec((1, E), lambda i: (0, 0)),
        ],
        out_specs=pl.BlockSpec((BB, P, E), lambda i: (i, 0, 0)),
        out_shape=jax.ShapeDtypeStruct((B, P, E), jnp.float32),
    )(masked_positions, sequence_data, W,
      b.reshape(1, E), gamma.reshape(1, E), beta.reshape(1, E))


def _sc_body(emb_hbm, cand_hbm, lm_hbm, out_hbm, idx_v, rows_v, lm_v, out_v,
             isem0, isem1, gsem0, gsem1, lsem0, lsem1, osem0, osem1):
    wid = lax.axis_index("s") * NC + lax.axis_index("c")
    w_base = wid * PER_W
    lanes = lax.iota(jnp.int32, 16)
    isems = (isem0, isem1)
    gsems = (gsem0, gsem1)
    lsems = (lsem0, lsem1)
    osems = (osem0, osem1)

    def idx_copy(c, b):
        base = w_base + c * G
        return pltpu.make_async_copy(
            cand_hbm.at[pl.ds(base * K_PAD, IDX_N)], idx_v.at[b], isems[b])

    def gather_copy(b):
        return pltpu.make_async_copy(
            emb_hbm.at[idx_v.at[b]], rows_v.at[b], gsems[b])

    def lm_copy(c, b):
        base = w_base + c * G
        return pltpu.make_async_copy(
            lm_hbm.at[pl.ds(base, G)], lm_v.at[b], lsems[b])

    def out_copy(c, b):
        base = w_base + c * G
        return pltpu.make_async_copy(
            out_v.at[b], out_hbm.at[pl.ds(base, G)], osems[b])

    def compute(c, b):
        for g in range(G):
            l0 = lm_v[b, g, pl.ds(0, 16)]
            l1 = lm_v[b, g, pl.ds(16, 16)]
            l2 = lm_v[b, g, pl.ds(32, 16)]
            l3 = lm_v[b, g, pl.ds(48, 16)]
            for t in range(4):
                acc = jnp.zeros((16,), jnp.float32)
                for m in range(min(16, K - 16 * t)):
                    r = g * K_PAD + 16 * t + m
                    prod = (rows_v[b, r, pl.ds(0, 16)] * l0
                            + rows_v[b, r, pl.ds(16, 16)] * l1
                            + rows_v[b, r, pl.ds(32, 16)] * l2
                            + rows_v[b, r, pl.ds(48, 16)] * l3)
                    acc = jnp.where(lanes == m, jnp.sum(prod), acc)
                out_v[b, g, pl.ds(16 * t, 16)] = acc

    def half(c, b):
        idx_copy(c + 1, 1 - b).wait()
        gather_copy(1 - b).start()
        lm_copy(c + 1, 1 - b).start()
        lm_copy(c, b).wait()
        gather_copy(b).wait()
        idx_copy(c + 2, b).start()
        @pl.when(c >= 2)
        def _():
            out_copy(c - 2, b).wait()
        compute(c, b)
        out_copy(c, b).start()

    d = idx_copy(0, 0)
    d.start()
    d.wait()
    gather_copy(0).start()
    idx_copy(1, 1).start()
    lm_copy(0, 0).start()

    def body(i2, carry):
        half(i2 * 2, 0)
        half(i2 * 2 + 1, 1)
        return carry

    lax.fori_loop(0, NCHUNK // 2, body, 0)

    gather_copy(0).wait()
    lm_copy(NCHUNK, 0).wait()
    idx_copy(NCHUNK + 1, 1).wait()
    out_copy(NCHUNK - 2, 0).wait()
    out_copy(NCHUNK - 1, 1).wait()


def _sc_score(embedding_table, cand_flat, lm_flat):
    mesh = plsc.VectorSubcoreMesh(core_axis_name="c", subcore_axis_name="s")
    kern = functools.partial(
        pl.kernel,
        out_type=jax.ShapeDtypeStruct((NPRED, 64), jnp.float32),
        mesh=mesh,
        scratch_types=[
            pltpu.VMEM((2, IDX_N), jnp.int32),
            pltpu.VMEM((2, IDX_N, E), jnp.float32),
            pltpu.VMEM((2, G, E), jnp.float32),
            pltpu.VMEM((2, G, 64), jnp.float32),
            pltpu.SemaphoreType.DMA,
            pltpu.SemaphoreType.DMA,
            pltpu.SemaphoreType.DMA,
            pltpu.SemaphoreType.DMA,
            pltpu.SemaphoreType.DMA,
            pltpu.SemaphoreType.DMA,
            pltpu.SemaphoreType.DMA,
            pltpu.SemaphoreType.DMA,
        ],
        compiler_params=pltpu.CompilerParams(
            needs_layout_passes=False, use_tc_tiling_on_sc=False),
    )(_sc_body)
    return kern(embedding_table, cand_flat, lm_flat)


def kernel(sequence_data, masked_positions, candidate_sets, embedding_table, W, b, gamma, beta):
    lm = _tc_lm(masked_positions, sequence_data, W, b, gamma, beta)
    lm_flat = lm.reshape(NPRED, E)
    lm_flat = jnp.concatenate(
        [lm_flat, jnp.zeros((8, E), jnp.float32)], axis=0)
    cand = candidate_sets.reshape(NPRED, K).astype(jnp.int32)
    cand_pad = jnp.concatenate([cand, cand[:, : K_PAD - K]], axis=1)
    cand_pad = jnp.concatenate(
        [cand_pad, jnp.zeros((8, K_PAD), jnp.int32)], axis=0)
    out = _sc_score(embedding_table, cand_pad.reshape(-1), lm_flat)
    return out[:, :K].reshape(B, P, K)
